# Initial kernel scaffold; baseline (speedup 1.0000x reference)
import jax
import jax.numpy as jnp
from jax.experimental import pallas as pl


def kernel(voxel_features, voxel_coords):
    raise NotImplementedError("write your pallas kernel here")


# speedup vs baseline: 2.2870x; 2.2870x over previous
import functools

import jax
import jax.numpy as jnp
from jax import lax
from jax.experimental import pallas as pl
from jax.experimental.pallas import tpu as pltpu
from jax.experimental.pallas import tpu_sc as plsc

R = 32
V = R * R * R
B = 4
C = 256
N = 16384
L = 16
NC = 2
NS = 16
NW = NC * NS
PW = (B * N) // NW
S = 16
G = 128
SPG = G // S
NGRP = PW // G
WPB = N // PW


def _body(table, coords, out, coords_v, idx_v, w_v, rows_v, out_v):
    ci = lax.axis_index("c")
    si = lax.axis_index("s")
    wid = si * NC + ci
    b = wid // WPB
    pbase = (wid % WPB) * PW

    pltpu.sync_copy(coords.at[b, :, pl.ds(pbase, PW)], coords_v)

    ramp = lax.iota(jnp.int32, L)
    d1 = [j * S + ramp for j in range(8)]

    def prep(p0):
        x = coords_v[0, pl.ds(p0, S)]
        y = coords_v[1, pl.ds(p0, S)]
        z = coords_v[2, pl.ds(p0, S)]
        x = jnp.minimum(jnp.maximum(x, 0.0), float(R - 1))
        y = jnp.minimum(jnp.maximum(y, 0.0), float(R - 1))
        z = jnp.minimum(jnp.maximum(z, 0.0), float(R - 1))
        x0 = x.astype(jnp.int32)
        y0 = y.astype(jnp.int32)
        z0 = z.astype(jnp.int32)
        wx = x - x0.astype(jnp.float32)
        wy = y - y0.astype(jnp.float32)
        wz = z - z0.astype(jnp.float32)
        x1 = jnp.minimum(x0 + 1, R - 1)
        y1 = jnp.minimum(y0 + 1, R - 1)
        z1 = jnp.minimum(z0 + 1, R - 1)
        gx = 1.0 - wx
        gy = 1.0 - wy
        gz = 1.0 - wz

        hx0 = x0 * (R * R) + b * V
        hx1 = x1 * (R * R) + b * V
        hy0 = y0 * R
        hy1 = y1 * R
        c00 = hx0 + hy0
        c01 = hx0 + hy1
        c10 = hx1 + hy0
        c11 = hx1 + hy1
        idx8 = [c00 + z0, c00 + z1, c01 + z0, c01 + z1,
                c10 + z0, c10 + z1, c11 + z0, c11 + z1]
        a00 = gx * gy
        a01 = gx * wy
        a10 = wx * gy
        a11 = wx * wy
        w8 = [a00 * gz, a00 * wz, a01 * gz, a01 * wz,
              a10 * gz, a10 * wz, a11 * gz, a11 * wz]
        for j in range(8):
            idx_v[pl.ds(j * S, S)] = idx8[j]
            w_v[j, :] = w8[j]

    def compute(col0):
        wv = [w_v[j, :] for j in range(8)]

        def cbody(c, _):
            d2 = jnp.full((L,), c, jnp.int32)
            acc = wv[0] * plsc.load_gather(rows_v, [d1[0], d2])
            for j in range(1, 8):
                acc = acc + wv[j] * plsc.load_gather(rows_v, [d1[j], d2])
            out_v[c, pl.ds(col0, S)] = acc
            return 0

        lax.fori_loop(0, C, cbody, 0, unroll=2)

    def group(g, _):
        def sub(s, _):
            p0 = g * G + s * S
            prep(p0)
            pltpu.sync_copy(table.at[idx_v], rows_v)
            compute(s * S)
            return 0

        lax.fori_loop(0, SPG, sub, 0)
        pltpu.sync_copy(out_v, out.at[b, :, pl.ds(pbase + g * G, G)])
        return 0

    lax.fori_loop(0, NGRP, group, 0)


_devox = pl.kernel(
    _body,
    out_type=jax.ShapeDtypeStruct((B, C, N), jnp.float32),
    mesh=plsc.VectorSubcoreMesh(
        core_axis_name="c", subcore_axis_name="s", num_cores=NC, num_subcores=NS
    ),
    compiler_params=pltpu.CompilerParams(needs_layout_passes=False),
    scratch_types=[
        pltpu.VMEM((3, PW), jnp.float32),
        pltpu.VMEM((8 * S,), jnp.int32),
        pltpu.VMEM((8, S), jnp.float32),
        pltpu.VMEM((8 * S, C), jnp.float32),
        pltpu.VMEM((C, G), jnp.float32),
    ],
)


@jax.jit
def kernel(voxel_features, voxel_coords):
    table = (
        voxel_features.reshape(B, C, V).transpose(0, 2, 1).reshape(B * V, C)
    )
    return _devox(table, voxel_coords)
